# baseline (speedup 1.0000x reference)
import functools

import jax
import jax.numpy as jnp
from jax import lax
from jax.experimental import pallas as pl
from jax.experimental.pallas import tpu as pltpu
from jax.experimental.pallas import tpu_sc as plsc

FIELDS = 8
B = 16384
V = 100000
D = 32
ND = 64
P = 128

_NC = 2
_NS = 16
_NW = _NC * _NS
_BPW = B // _NW


def _make_sc_gather(nf):
    mesh = plsc.VectorSubcoreMesh(core_axis_name="c", subcore_axis_name="s")

    @functools.partial(
        pl.kernel,
        mesh=mesh,
        out_type=jax.ShapeDtypeStruct((nf, B, D), jnp.float32),
        scratch_types=[
            pltpu.VMEM((_BPW,), jnp.int32),
            pltpu.VMEM((_BPW, D), jnp.float32),
            pltpu.SemaphoreType.DMA,
        ],
        compiler_params=pltpu.CompilerParams(use_tc_tiling_on_sc=False),
    )
    def sc_gather(*refs):
        idxs = refs[:nf]
        tabs = refs[nf:2 * nf]
        out = refs[2 * nf]
        idx_v, rows_v, sem = refs[2 * nf + 1:]
        wid = lax.axis_index("s") * _NC + lax.axis_index("c")
        base = wid * _BPW
        for f in range(nf):
            pltpu.sync_copy(idxs[f].at[pl.ds(base, _BPW)], idx_v)
            pltpu.async_copy(tabs[f].at[idx_v], rows_v, sem).wait()
            pltpu.sync_copy(rows_v, out.at[f].at[pl.ds(base, _BPW)])

    return sc_gather


_SC_GATHER_Q = _make_sc_gather(FIELDS // 4)

_BS = 1024


def _tc_body(emb_a_ref, emb_b_ref, emb_c_ref, emb_d_ref,
             num_ref, g_ref, be_ref, wn_ref, bn_ref,
             wcat_ref, wnum_ref, bf_ref, out_ref):
    x = num_ref[...]
    mu = jnp.mean(x, axis=-1, keepdims=True)
    var = jnp.mean((x - mu) ** 2, axis=-1, keepdims=True)
    xn = (x - mu) * lax.rsqrt(var + 1e-5) * g_ref[...] + be_ref[...]
    nf = jnp.maximum(
        jnp.dot(xn, wn_ref[...], preferred_element_type=jnp.float32)
        + bn_ref[...], 0.0)
    parts = []
    for r in (emb_a_ref, emb_b_ref, emb_c_ref, emb_d_ref):
        e = r[...]
        parts.extend([e[f] for f in range(FIELDS // 4)])
    cat = jnp.concatenate(parts, axis=-1)
    acc = jnp.dot(cat, wcat_ref[...], preferred_element_type=jnp.float32)
    acc = acc + jnp.dot(nf, wnum_ref[...], preferred_element_type=jnp.float32)
    out_ref[...] = jnp.maximum(acc + bf_ref[...], 0.0)


def kernel(idx_0, idx_1, idx_2, idx_3, idx_4, idx_5, idx_6, idx_7,
           numeric_input,
           table_0, table_1, table_2, table_3, table_4, table_5, table_6,
           table_7, ln_gamma, ln_beta, W_num, b_num, W_final, b_final):
    emb_a = _SC_GATHER_Q(idx_0, idx_1, table_0, table_1)
    emb_b = _SC_GATHER_Q(idx_2, idx_3, table_2, table_3)
    emb_c = _SC_GATHER_Q(idx_4, idx_5, table_4, table_5)
    emb_d = _SC_GATHER_Q(idx_6, idx_7, table_6, table_7)
    gam = ln_gamma.reshape(1, ND)
    bet = ln_beta.reshape(1, ND)
    bn = b_num.reshape(1, P)
    bf = b_final.reshape(1, P)
    wcat = W_final[:FIELDS * D]
    wnum = W_final[FIELDS * D:]
    out = pl.pallas_call(
        _tc_body,
        grid=(B // _BS,),
        in_specs=[
            pl.BlockSpec((FIELDS // 4, _BS, D), lambda i: (0, i, 0)),
            pl.BlockSpec((FIELDS // 4, _BS, D), lambda i: (0, i, 0)),
            pl.BlockSpec((FIELDS // 4, _BS, D), lambda i: (0, i, 0)),
            pl.BlockSpec((FIELDS // 4, _BS, D), lambda i: (0, i, 0)),
            pl.BlockSpec((_BS, ND), lambda i: (i, 0)),
            pl.BlockSpec((1, ND), lambda i: (0, 0)),
            pl.BlockSpec((1, ND), lambda i: (0, 0)),
            pl.BlockSpec((ND, P), lambda i: (0, 0)),
            pl.BlockSpec((1, P), lambda i: (0, 0)),
            pl.BlockSpec((FIELDS * D, P), lambda i: (0, 0)),
            pl.BlockSpec((P, P), lambda i: (0, 0)),
            pl.BlockSpec((1, P), lambda i: (0, 0)),
        ],
        out_specs=pl.BlockSpec((_BS, P), lambda i: (i, 0)),
        out_shape=jax.ShapeDtypeStruct((B, P), jnp.float32),
    )(emb_a, emb_b, emb_c, emb_d, numeric_input, gam, bet, W_num, bn,
      wcat, wnum, bf)
    return out


# speedup vs baseline: 1.0027x; 1.0027x over previous
import functools

import jax
import jax.numpy as jnp
from jax import lax
from jax.experimental import pallas as pl
from jax.experimental.pallas import tpu as pltpu
from jax.experimental.pallas import tpu_sc as plsc

FIELDS = 8
B = 16384
V = 100000
D = 32
ND = 64
P = 128

_NC = 2
_NS = 16
_NW = _NC * _NS
_BPW = B // _NW


def _make_sc_gather(nf):
    mesh = plsc.VectorSubcoreMesh(core_axis_name="c", subcore_axis_name="s")

    @functools.partial(
        pl.kernel,
        mesh=mesh,
        out_type=jax.ShapeDtypeStruct((nf, B, D), jnp.float32),
        scratch_types=[
            pltpu.VMEM((_BPW,), jnp.int32),
            pltpu.VMEM((_BPW, D), jnp.float32),
            pltpu.SemaphoreType.DMA,
        ],
        compiler_params=pltpu.CompilerParams(use_tc_tiling_on_sc=False),
    )
    def sc_gather(*refs):
        idxs = refs[:nf]
        tabs = refs[nf:2 * nf]
        out = refs[2 * nf]
        idx_v, rows_v, sem = refs[2 * nf + 1:]
        wid = lax.axis_index("s") * _NC + lax.axis_index("c")
        base = wid * _BPW
        for f in range(nf):
            pltpu.sync_copy(idxs[f].at[pl.ds(base, _BPW)], idx_v)
            pltpu.async_copy(tabs[f].at[idx_v], rows_v, sem).wait()
            pltpu.sync_copy(rows_v, out.at[f].at[pl.ds(base, _BPW)])

    return sc_gather


_SC_GATHER_H = _make_sc_gather(FIELDS // 2)

_BS = 1024


def _tc_body(emb_a_ref, emb_b_ref, num_ref, g_ref, be_ref, wn_ref, bn_ref,
             wcat_ref, wnum_ref, bf_ref, out_ref):
    x = num_ref[...]
    mu = jnp.mean(x, axis=-1, keepdims=True)
    var = jnp.mean((x - mu) ** 2, axis=-1, keepdims=True)
    xn = (x - mu) * lax.rsqrt(var + 1e-5) * g_ref[...] + be_ref[...]
    nf = jnp.maximum(
        jnp.dot(xn, wn_ref[...], preferred_element_type=jnp.float32)
        + bn_ref[...], 0.0)
    ea = emb_a_ref[...]
    eb = emb_b_ref[...]
    cat = jnp.concatenate(
        [ea[f] for f in range(FIELDS // 2)]
        + [eb[f] for f in range(FIELDS // 2)], axis=-1)
    acc = jnp.dot(cat, wcat_ref[...], preferred_element_type=jnp.float32)
    acc = acc + jnp.dot(nf, wnum_ref[...], preferred_element_type=jnp.float32)
    out_ref[...] = jnp.maximum(acc + bf_ref[...], 0.0)


def kernel(idx_0, idx_1, idx_2, idx_3, idx_4, idx_5, idx_6, idx_7,
           numeric_input,
           table_0, table_1, table_2, table_3, table_4, table_5, table_6,
           table_7, ln_gamma, ln_beta, W_num, b_num, W_final, b_final):
    emb_a = _SC_GATHER_H(idx_0, idx_1, idx_2, idx_3,
                         table_0, table_1, table_2, table_3)
    emb_b = _SC_GATHER_H(idx_4, idx_5, idx_6, idx_7,
                         table_4, table_5, table_6, table_7)
    gam = ln_gamma.reshape(1, ND)
    bet = ln_beta.reshape(1, ND)
    bn = b_num.reshape(1, P)
    bf = b_final.reshape(1, P)
    wcat = W_final[:FIELDS * D]
    wnum = W_final[FIELDS * D:]
    out = pl.pallas_call(
        _tc_body,
        grid=(B // _BS,),
        in_specs=[
            pl.BlockSpec((FIELDS // 2, _BS, D), lambda i: (0, i, 0)),
            pl.BlockSpec((FIELDS // 2, _BS, D), lambda i: (0, i, 0)),
            pl.BlockSpec((_BS, ND), lambda i: (i, 0)),
            pl.BlockSpec((1, ND), lambda i: (0, 0)),
            pl.BlockSpec((1, ND), lambda i: (0, 0)),
            pl.BlockSpec((ND, P), lambda i: (0, 0)),
            pl.BlockSpec((1, P), lambda i: (0, 0)),
            pl.BlockSpec((FIELDS * D, P), lambda i: (0, 0)),
            pl.BlockSpec((P, P), lambda i: (0, 0)),
            pl.BlockSpec((1, P), lambda i: (0, 0)),
        ],
        out_specs=pl.BlockSpec((_BS, P), lambda i: (i, 0)),
        out_shape=jax.ShapeDtypeStruct((B, P), jnp.float32),
    )(emb_a, emb_b, numeric_input, gam, bet, W_num, bn, wcat, wnum, bf)
    return out


# speedup vs baseline: 1.1380x; 1.1349x over previous
import functools

import jax
import jax.numpy as jnp
from jax import lax
from jax.experimental import pallas as pl
from jax.experimental.pallas import tpu as pltpu
from jax.experimental.pallas import tpu_sc as plsc

FIELDS = 8
B = 16384
V = 100000
D = 32
ND = 64
P = 128

_NC = 2
_NS = 16
_NW = _NC * _NS
_BPW = B // _NW


def _make_sc_gather(nf):
    mesh = plsc.VectorSubcoreMesh(core_axis_name="c", subcore_axis_name="s")

    @functools.partial(
        pl.kernel,
        mesh=mesh,
        out_type=jax.ShapeDtypeStruct((B, nf * D), jnp.float32),
        scratch_types=[
            pltpu.VMEM((_BPW,), jnp.int32),
            pltpu.VMEM((_BPW, D), jnp.float32),
            pltpu.SemaphoreType.DMA,
        ],
        compiler_params=pltpu.CompilerParams(use_tc_tiling_on_sc=False),
    )
    def sc_gather(*refs):
        idxs = refs[:nf]
        tabs = refs[nf:2 * nf]
        out = refs[2 * nf]
        idx_v, rows_v, sem = refs[2 * nf + 1:]
        wid = lax.axis_index("s") * _NC + lax.axis_index("c")
        base = wid * _BPW
        for f in range(nf):
            pltpu.sync_copy(idxs[f].at[pl.ds(base, _BPW)], idx_v)
            pltpu.async_copy(tabs[f].at[idx_v], rows_v, sem).wait()
            pltpu.sync_copy(rows_v,
                            out.at[pl.ds(base, _BPW), pl.ds(f * D, D)])

    return sc_gather


_SC_GATHER_H = _make_sc_gather(FIELDS // 2)

_BS = 1024


def _tc_body(emb_a_ref, emb_b_ref, num_ref, g_ref, be_ref, wn_ref, bn_ref,
             wcat_ref, wnum_ref, bf_ref, out_ref):
    x = num_ref[...]
    mu = jnp.mean(x, axis=-1, keepdims=True)
    var = jnp.mean((x - mu) ** 2, axis=-1, keepdims=True)
    xn = (x - mu) * lax.rsqrt(var + 1e-5) * g_ref[...] + be_ref[...]
    nf = jnp.maximum(
        jnp.dot(xn, wn_ref[...], preferred_element_type=jnp.float32)
        + bn_ref[...], 0.0)
    ea = emb_a_ref[...]
    eb = emb_b_ref[...]
    half = FIELDS * D // 2
    acc = jnp.dot(ea, wcat_ref[:half], preferred_element_type=jnp.float32)
    acc = acc + jnp.dot(eb, wcat_ref[half:],
                        preferred_element_type=jnp.float32)
    acc = acc + jnp.dot(nf, wnum_ref[...], preferred_element_type=jnp.float32)
    out_ref[...] = jnp.maximum(acc + bf_ref[...], 0.0)


def kernel(idx_0, idx_1, idx_2, idx_3, idx_4, idx_5, idx_6, idx_7,
           numeric_input,
           table_0, table_1, table_2, table_3, table_4, table_5, table_6,
           table_7, ln_gamma, ln_beta, W_num, b_num, W_final, b_final):
    emb_a = _SC_GATHER_H(idx_0, idx_1, idx_2, idx_3,
                         table_0, table_1, table_2, table_3)
    emb_b = _SC_GATHER_H(idx_4, idx_5, idx_6, idx_7,
                         table_4, table_5, table_6, table_7)
    gam = ln_gamma.reshape(1, ND)
    bet = ln_beta.reshape(1, ND)
    bn = b_num.reshape(1, P)
    bf = b_final.reshape(1, P)
    wcat = W_final[:FIELDS * D]
    wnum = W_final[FIELDS * D:]
    out = pl.pallas_call(
        _tc_body,
        grid=(B // _BS,),
        in_specs=[
            pl.BlockSpec((_BS, FIELDS * D // 2), lambda i: (i, 0)),
            pl.BlockSpec((_BS, FIELDS * D // 2), lambda i: (i, 0)),
            pl.BlockSpec((_BS, ND), lambda i: (i, 0)),
            pl.BlockSpec((1, ND), lambda i: (0, 0)),
            pl.BlockSpec((1, ND), lambda i: (0, 0)),
            pl.BlockSpec((ND, P), lambda i: (0, 0)),
            pl.BlockSpec((1, P), lambda i: (0, 0)),
            pl.BlockSpec((FIELDS * D, P), lambda i: (0, 0)),
            pl.BlockSpec((P, P), lambda i: (0, 0)),
            pl.BlockSpec((1, P), lambda i: (0, 0)),
        ],
        out_specs=pl.BlockSpec((_BS, P), lambda i: (i, 0)),
        out_shape=jax.ShapeDtypeStruct((B, P), jnp.float32),
    )(emb_a, emb_b, numeric_input, gam, bet, W_num, bn, wcat, wnum, bf)
    return out
